# Initial kernel scaffold; baseline (speedup 1.0000x reference)
import jax
import jax.numpy as jnp
from jax.experimental import pallas as pl


def kernel(x, edge_index, W, b):
    raise NotImplementedError("write your pallas kernel here")


# speedup vs baseline: 12.6713x; 12.6713x over previous
import functools

import jax
import jax.numpy as jnp
from jax import lax
from jax.experimental import pallas as pl
from jax.experimental.pallas import tpu as pltpu
from jax.experimental.pallas import tpu_sc as plsc


def _tables_body(x_ref, w_ref, bias_ref, y1_ref, y2_ref, *, C):
    xb = x_ref[0]
    w1 = w_ref[:, :C]
    w2 = w_ref[:, C:]
    a1 = w1 - w2
    y1 = lax.dot_general(xb, a1, (((0,), (1,)), ((), ())),
                         preferred_element_type=jnp.float32)
    y2 = lax.dot_general(xb, w2, (((0,), (1,)), ((), ())),
                         preferred_element_type=jnp.float32)
    y1_ref[0] = y1 + bias_ref[0][None, :]
    y2_ref[0] = y2


def _make_tables(xs, W, bias2):
    B, C, N = xs.shape
    O = W.shape[0]
    return pl.pallas_call(
        functools.partial(_tables_body, C=C),
        grid=(B,),
        in_specs=[
            pl.BlockSpec((1, C, N), lambda i: (i, 0, 0)),
            pl.BlockSpec((O, 2 * C), lambda i: (0, 0)),
            pl.BlockSpec((1, O), lambda i: (0, 0)),
        ],
        out_specs=[
            pl.BlockSpec((1, N, O), lambda i: (i, 0, 0)),
            pl.BlockSpec((1, N, O), lambda i: (i, 0, 0)),
        ],
        out_shape=[
            jax.ShapeDtypeStruct((B, N, O), jnp.float32),
            jax.ShapeDtypeStruct((B, N, O), jnp.float32),
        ],
    )(xs, W, bias2)


def _edge_max_call(y1f, y2f, g1, g0, *, K, O, CH, NCH, PPW, NC):
    R, _ = y1f.shape
    IDXC = CH * K
    LAN = 16
    NJ = O // LAN

    mesh = plsc.VectorSubcoreMesh(core_axis_name="c", subcore_axis_name="s")

    @functools.partial(
        pl.kernel,
        out_type=jax.ShapeDtypeStruct((R, O), jnp.float32),
        mesh=mesh,
        compiler_params=pltpu.CompilerParams(use_tc_tiling_on_sc=False),
        scratch_types=[
            pltpu.VMEM((IDXC,), jnp.int32),
            pltpu.VMEM((IDXC,), jnp.int32),
            pltpu.VMEM((IDXC, O), jnp.float32),
            pltpu.VMEM((IDXC, O), jnp.float32),
            pltpu.VMEM((CH, O), jnp.float32),
            pltpu.SemaphoreType.DMA,
            pltpu.SemaphoreType.DMA,
        ],
    )
    def sc_kernel(y1_hbm, y2_hbm, g1_hbm, g0_hbm, out_hbm,
                  idx1_v, idx0_v, rows1_v, rows0_v, acc_v, sem1, sem0):
        wid = lax.axis_index("s") * NC + lax.axis_index("c")

        def chunk(ch, carry):
            pbase = wid * PPW + ch * CH
            ibase = pbase * K
            pltpu.sync_copy(g1_hbm.at[pl.ds(ibase, IDXC)], idx1_v)
            pltpu.sync_copy(g0_hbm.at[pl.ds(ibase, IDXC)], idx0_v)
            cp1 = pltpu.async_copy(y1_hbm.at[idx1_v], rows1_v, sem1)
            cp0 = pltpu.async_copy(y2_hbm.at[idx0_v], rows0_v, sem0)
            cp1.wait()
            cp0.wait()

            def pair(p, c2):
                row = p * K
                for j in range(NJ):
                    sl = pl.ds(j * LAN, LAN)

                    def kstep(k, acc):
                        v = rows1_v[row + k, sl] + rows0_v[row + k, sl]
                        return jnp.maximum(acc, v)

                    a0 = rows1_v[row, sl] + rows0_v[row, sl]
                    a = lax.fori_loop(1, K, kstep, a0)
                    acc_v[p, sl] = jnp.maximum(a, 0.0)
                return c2

            lax.fori_loop(0, CH, pair, 0)
            pltpu.sync_copy(acc_v, out_hbm.at[pl.ds(pbase, CH), :])
            return carry

        lax.fori_loop(0, NCH, chunk, 0)

    return sc_kernel(y1f, y2f, g1, g0)


def _transpose_body(y_ref, o_ref):
    o_ref[0] = y_ref[0].T


def _transpose(y, B, N, O):
    return pl.pallas_call(
        _transpose_body,
        grid=(B,),
        in_specs=[pl.BlockSpec((1, N, O), lambda i: (i, 0, 0))],
        out_specs=pl.BlockSpec((1, O, N), lambda i: (i, 0, 0)),
        out_shape=jax.ShapeDtypeStruct((B, O, N), jnp.float32),
    )(y)


def kernel(x, edge_index, W, b):
    B, C, N, _ = x.shape
    K = edge_index.shape[-1]
    O = W.shape[0]

    xs = x[:, :, :, 0]
    bias2 = b[None, :]

    idx = edge_index.astype(jnp.int32)
    base = (jnp.arange(B, dtype=jnp.int32) * N)[None, :, None, None]
    g = (idx + base).reshape(2, B * N * K)

    y1, y2 = _make_tables(xs, W, bias2)
    y1f = y1.reshape(B * N, O)
    y2f = y2.reshape(B * N, O)

    info = plsc.get_sparse_core_info()
    NC, NS = info.num_cores, info.num_subcores
    NW = NC * NS
    PAIRS = B * N
    PPW = PAIRS // NW
    CH = 8
    NCH = PPW // CH

    out_f = _edge_max_call(y1f, y2f, g[1], g[0],
                           K=K, O=O, CH=CH, NCH=NCH, PPW=PPW, NC=NC)

    out = _transpose(out_f.reshape(B, N, O), B, N, O)
    return out[..., None]


# speedup vs baseline: 25.2944x; 1.9962x over previous
import functools

import jax
import jax.numpy as jnp
from jax import lax
from jax.experimental import pallas as pl
from jax.experimental.pallas import tpu as pltpu
from jax.experimental.pallas import tpu_sc as plsc


def _tables_body(x_ref, w_ref, bias_ref, y1_ref, y2_ref, *, C):
    xb = x_ref[0]
    w1 = w_ref[:, :C]
    w2 = w_ref[:, C:]
    a1 = w1 - w2
    y1 = lax.dot_general(xb, a1, (((0,), (1,)), ((), ())),
                         preferred_element_type=jnp.float32)
    y2 = lax.dot_general(xb, w2, (((0,), (1,)), ((), ())),
                         preferred_element_type=jnp.float32)
    y1_ref[0] = y1 + bias_ref[0][None, :]
    y2_ref[0] = y2


def _make_tables(xs, W, bias2):
    B, C, N = xs.shape
    O = W.shape[0]
    return pl.pallas_call(
        functools.partial(_tables_body, C=C),
        grid=(B,),
        in_specs=[
            pl.BlockSpec((1, C, N), lambda i: (i, 0, 0)),
            pl.BlockSpec((O, 2 * C), lambda i: (0, 0)),
            pl.BlockSpec((1, O), lambda i: (0, 0)),
        ],
        out_specs=[
            pl.BlockSpec((1, N, O), lambda i: (i, 0, 0)),
            pl.BlockSpec((1, N, O), lambda i: (i, 0, 0)),
        ],
        out_shape=[
            jax.ShapeDtypeStruct((B, N, O), jnp.float32),
            jax.ShapeDtypeStruct((B, N, O), jnp.float32),
        ],
    )(xs, W, bias2)


def _edge_max_call(y1f, y2f, g1, g0, *, K, O, CH, NCH, PPW, NC):
    R, _ = y1f.shape
    IDXC = CH * K
    LAN = 16
    NJ = O // LAN
    NBUF = 2

    mesh = plsc.VectorSubcoreMesh(core_axis_name="c", subcore_axis_name="s")

    @functools.partial(
        pl.kernel,
        out_type=jax.ShapeDtypeStruct((R, O), jnp.float32),
        mesh=mesh,
        compiler_params=pltpu.CompilerParams(use_tc_tiling_on_sc=False),
        scratch_types=[
            pltpu.VMEM((NCH, IDXC), jnp.int32),
            pltpu.VMEM((NCH, IDXC), jnp.int32),
            [pltpu.VMEM((IDXC, O), jnp.float32) for _ in range(NBUF)],
            [pltpu.VMEM((IDXC, O), jnp.float32) for _ in range(NBUF)],
            [pltpu.VMEM((CH, O), jnp.float32) for _ in range(NBUF)],
            [pltpu.SemaphoreType.DMA for _ in range(NBUF)],
            [pltpu.SemaphoreType.DMA for _ in range(NBUF)],
            [pltpu.SemaphoreType.DMA for _ in range(NBUF)],
        ],
    )
    def sc_kernel(y1_hbm, y2_hbm, g1_hbm, g0_hbm, out_hbm,
                  idx1_v, idx0_v, rows1, rows0, acc, sem1, sem0, semo):
        wid = lax.axis_index("s") * NC + lax.axis_index("c")

        pltpu.sync_copy(g1_hbm.at[wid], idx1_v)
        pltpu.sync_copy(g0_hbm.at[wid], idx0_v)

        def start_gathers(ch, b):
            pltpu.async_copy(y1_hbm.at[idx1_v.at[ch]], rows1[b], sem1[b])
            pltpu.async_copy(y2_hbm.at[idx0_v.at[ch]], rows0[b], sem0[b])

        def wait_gathers(b):
            pltpu.make_async_copy(y1_hbm.at[idx1_v.at[0]], rows1[b],
                                  sem1[b]).wait()
            pltpu.make_async_copy(y2_hbm.at[idx0_v.at[0]], rows0[b],
                                  sem0[b]).wait()

        def wait_out(b):
            pltpu.make_async_copy(acc[b], out_hbm.at[pl.ds(0, CH), :],
                                  semo[b]).wait()

        for b in range(NBUF):
            start_gathers(b, b)

        def compute_chunk(b):
            def pair(p, c2):
                row = p * K
                for j in range(NJ):
                    sl = pl.ds(j * LAN, LAN)
                    m = None
                    for k in range(K):
                        v = rows1[b][row + k, sl] + rows0[b][row + k, sl]
                        m = v if m is None else jnp.maximum(m, v)
                    acc[b][p, sl] = jnp.maximum(m, 0.0)
                return c2

            lax.fori_loop(0, CH, pair, 0, unroll=2)

        def super_step(si, carry):
            for b in range(NBUF):
                ch = si * NBUF + b
                wait_gathers(b)

                @pl.when(ch >= NBUF)
                def _():
                    wait_out(b)

                compute_chunk(b)
                pltpu.async_copy(acc[b], out_hbm.at[pl.ds(wid * PPW + ch * CH, CH), :],
                                 semo[b])

                @pl.when(ch + NBUF < NCH)
                def _():
                    start_gathers(ch + NBUF, b)
            return carry

        lax.fori_loop(0, NCH // NBUF, super_step, 0)
        for b in range(NBUF):
            wait_out(b)

    return sc_kernel(y1f, y2f, g1, g0)


def _transpose_body(y_ref, o_ref):
    o_ref[0] = y_ref[0].T


def _transpose(y, B, N, O):
    return pl.pallas_call(
        _transpose_body,
        grid=(B,),
        in_specs=[pl.BlockSpec((1, N, O), lambda i: (i, 0, 0))],
        out_specs=pl.BlockSpec((1, O, N), lambda i: (i, 0, 0)),
        out_shape=jax.ShapeDtypeStruct((B, O, N), jnp.float32),
    )(y)


def kernel(x, edge_index, W, b):
    B, C, N, _ = x.shape
    K = edge_index.shape[-1]
    O = W.shape[0]

    xs = x[:, :, :, 0]
    bias2 = b[None, :]

    idx = edge_index.astype(jnp.int32)
    base = (jnp.arange(B, dtype=jnp.int32) * N)[None, :, None, None]
    gflat = (idx + base).reshape(2, B * N * K)

    y1, y2 = _make_tables(xs, W, bias2)
    y1f = y1.reshape(B * N, O)
    y2f = y2.reshape(B * N, O)

    info = plsc.get_sparse_core_info()
    NC, NS = info.num_cores, info.num_subcores
    NW = NC * NS
    PAIRS = B * N
    PPW = PAIRS // NW
    CH = 8
    NCH = PPW // CH
    g = gflat.reshape(2, NW, NCH, CH * K)

    out_f = _edge_max_call(y1f, y2f, g[1], g[0],
                           K=K, O=O, CH=CH, NCH=NCH, PPW=PPW, NC=NC)

    out = _transpose(out_f.reshape(B, N, O), B, N, O)
    return out[..., None]


# speedup vs baseline: 25.3833x; 1.0035x over previous
import functools

import jax
import jax.numpy as jnp
from jax import lax
from jax.experimental import pallas as pl
from jax.experimental.pallas import tpu as pltpu
from jax.experimental.pallas import tpu_sc as plsc


def _tables_body(x_ref, w_ref, bias_ref, y1_ref, y2_ref, *, C):
    xb = x_ref[0]
    w1 = w_ref[:, :C]
    w2 = w_ref[:, C:]
    a1 = w1 - w2
    y1 = lax.dot_general(xb, a1, (((0,), (1,)), ((), ())),
                         preferred_element_type=jnp.float32)
    y2 = lax.dot_general(xb, w2, (((0,), (1,)), ((), ())),
                         preferred_element_type=jnp.float32)
    y1_ref[...] = y1 + bias_ref[0][None, :]
    y2_ref[...] = y2


def _make_tables(xs, W, bias2):
    B, C, N = xs.shape
    O = W.shape[0]
    return pl.pallas_call(
        functools.partial(_tables_body, C=C),
        grid=(B,),
        in_specs=[
            pl.BlockSpec((1, C, N), lambda i: (i, 0, 0)),
            pl.BlockSpec((O, 2 * C), lambda i: (0, 0)),
            pl.BlockSpec((1, O), lambda i: (0, 0)),
        ],
        out_specs=[
            pl.BlockSpec((N, O), lambda i: (i, 0)),
            pl.BlockSpec((N, O), lambda i: (i, 0)),
        ],
        out_shape=[
            jax.ShapeDtypeStruct((B * N, O), jnp.float32),
            jax.ShapeDtypeStruct((B * N, O), jnp.float32),
        ],
    )(xs, W, bias2)


def _edge_max_call(y1f, y2f, g1, g0, *, K, O, CH, NCH, PPW, NC):
    R, _ = y1f.shape
    IDXC = CH * K
    LAN = 16
    NJ = O // LAN
    NBUF = 2

    mesh = plsc.VectorSubcoreMesh(core_axis_name="c", subcore_axis_name="s")

    @functools.partial(
        pl.kernel,
        out_type=jax.ShapeDtypeStruct((R, O), jnp.float32),
        mesh=mesh,
        compiler_params=pltpu.CompilerParams(use_tc_tiling_on_sc=False),
        scratch_types=[
            pltpu.VMEM((NCH, IDXC), jnp.int32),
            pltpu.VMEM((NCH, IDXC), jnp.int32),
            [pltpu.VMEM((IDXC, O), jnp.float32) for _ in range(NBUF)],
            [pltpu.VMEM((IDXC, O), jnp.float32) for _ in range(NBUF)],
            [pltpu.VMEM((CH, O), jnp.float32) for _ in range(NBUF)],
            [pltpu.SemaphoreType.DMA for _ in range(NBUF)],
            [pltpu.SemaphoreType.DMA for _ in range(NBUF)],
            [pltpu.SemaphoreType.DMA for _ in range(NBUF)],
        ],
    )
    def sc_kernel(y1_hbm, y2_hbm, g1_hbm, g0_hbm, out_hbm,
                  idx1_v, idx0_v, rows1, rows0, acc, sem1, sem0, semo):
        wid = lax.axis_index("s") * NC + lax.axis_index("c")

        pltpu.sync_copy(g1_hbm.at[wid], idx1_v)
        pltpu.sync_copy(g0_hbm.at[wid], idx0_v)

        def start_gathers(ch, b):
            pltpu.async_copy(y1_hbm.at[idx1_v.at[ch]], rows1[b], sem1[b])
            pltpu.async_copy(y2_hbm.at[idx0_v.at[ch]], rows0[b], sem0[b])

        def wait_gathers(b):
            pltpu.make_async_copy(y1_hbm.at[idx1_v.at[0]], rows1[b],
                                  sem1[b]).wait()
            pltpu.make_async_copy(y2_hbm.at[idx0_v.at[0]], rows0[b],
                                  sem0[b]).wait()

        def wait_out(b):
            pltpu.make_async_copy(acc[b], out_hbm.at[pl.ds(0, CH), :],
                                  semo[b]).wait()

        for b in range(NBUF):
            start_gathers(b, b)

        def compute_chunk(b):
            def pair(p, c2):
                row = p * K
                for j in range(NJ):
                    sl = pl.ds(j * LAN, LAN)
                    m = None
                    for k in range(K):
                        v = rows1[b][row + k, sl] + rows0[b][row + k, sl]
                        m = v if m is None else jnp.maximum(m, v)
                    acc[b][p, sl] = jnp.maximum(m, 0.0)
                return c2

            lax.fori_loop(0, CH, pair, 0, unroll=2)

        def super_step(si, carry):
            for b in range(NBUF):
                ch = si * NBUF + b
                wait_gathers(b)

                @pl.when(ch >= NBUF)
                def _():
                    wait_out(b)

                compute_chunk(b)
                pltpu.async_copy(acc[b], out_hbm.at[pl.ds(wid * PPW + ch * CH, CH), :],
                                 semo[b])

                @pl.when(ch + NBUF < NCH)
                def _():
                    start_gathers(ch + NBUF, b)
            return carry

        lax.fori_loop(0, NCH // NBUF, super_step, 0)
        for b in range(NBUF):
            wait_out(b)

    return sc_kernel(y1f, y2f, g1, g0)


def _transpose_body(y_ref, o_ref):
    o_ref[0] = y_ref[...].astype(jnp.float32).T


def _transpose(y, B, N, O):
    return pl.pallas_call(
        _transpose_body,
        grid=(B,),
        in_specs=[pl.BlockSpec((N, O), lambda i: (i, 0))],
        out_specs=pl.BlockSpec((1, O, N), lambda i: (i, 0, 0)),
        out_shape=jax.ShapeDtypeStruct((B, O, N), jnp.float32),
    )(y)


def kernel(x, edge_index, W, b):
    B, C, N, _ = x.shape
    K = edge_index.shape[-1]
    O = W.shape[0]

    xs = x[:, :, :, 0]
    bias2 = b[None, :]

    idx = edge_index.astype(jnp.int32)
    base = (jnp.arange(B, dtype=jnp.int32) * N)[None, :, None, None]
    gflat = (idx + base).reshape(2, B * N * K)

    y1f, y2f = _make_tables(xs, W, bias2)

    info = plsc.get_sparse_core_info()
    NC, NS = info.num_cores, info.num_subcores
    NW = NC * NS
    PAIRS = B * N
    PPW = PAIRS // NW
    CH = 8
    NCH = PPW // CH
    g = gflat.reshape(2, NW, NCH, CH * K)

    out_f = _edge_max_call(y1f, y2f, g[1], g[0],
                           K=K, O=O, CH=CH, NCH=NCH, PPW=PPW, NC=NC)

    out = _transpose(out_f, B, N, O)
    return out[..., None]


# speedup vs baseline: 30.4138x; 1.1982x over previous
import functools

import jax
import jax.numpy as jnp
from jax import lax
from jax.experimental import pallas as pl
from jax.experimental.pallas import tpu as pltpu
from jax.experimental.pallas import tpu_sc as plsc


def _tables_body(x_ref, w_ref, bias_ref, y1_ref, y2_ref, *, C):
    xb = x_ref[0]
    w1 = w_ref[:, :C]
    w2 = w_ref[:, C:]
    a1 = w1 - w2
    y1 = lax.dot_general(xb, a1, (((0,), (1,)), ((), ())),
                         preferred_element_type=jnp.float32)
    y2 = lax.dot_general(xb, w2, (((0,), (1,)), ((), ())),
                         preferred_element_type=jnp.float32)
    y1_ref[...] = (y1 + bias_ref[0][None, :]).astype(jnp.bfloat16)
    y2_ref[...] = y2.astype(jnp.bfloat16)


def _make_tables(xs, W, bias2):
    B, C, N = xs.shape
    O = W.shape[0]
    return pl.pallas_call(
        functools.partial(_tables_body, C=C),
        grid=(B,),
        in_specs=[
            pl.BlockSpec((1, C, N), lambda i: (i, 0, 0)),
            pl.BlockSpec((O, 2 * C), lambda i: (0, 0)),
            pl.BlockSpec((1, O), lambda i: (0, 0)),
        ],
        out_specs=[
            pl.BlockSpec((N, O), lambda i: (i, 0)),
            pl.BlockSpec((N, O), lambda i: (i, 0)),
        ],
        out_shape=[
            jax.ShapeDtypeStruct((B * N, O), jnp.bfloat16),
            jax.ShapeDtypeStruct((B * N, O), jnp.bfloat16),
        ],
    )(xs, W, bias2)


def _edge_max_call(y1f, y2f, g1, g0, *, K, O, CH, NCH, PPW, NC):
    R, _ = y1f.shape
    IDXC = CH * K
    LAN = 32
    NJ = O // LAN
    NBUF = 2

    mesh = plsc.VectorSubcoreMesh(core_axis_name="c", subcore_axis_name="s")

    @functools.partial(
        pl.kernel,
        out_type=jax.ShapeDtypeStruct((R, O), jnp.bfloat16),
        mesh=mesh,
        compiler_params=pltpu.CompilerParams(use_tc_tiling_on_sc=False),
        scratch_types=[
            pltpu.VMEM((NCH, IDXC), jnp.int32),
            pltpu.VMEM((NCH, IDXC), jnp.int32),
            [pltpu.VMEM((IDXC, O), jnp.bfloat16) for _ in range(NBUF)],
            [pltpu.VMEM((IDXC, O), jnp.bfloat16) for _ in range(NBUF)],
            [pltpu.VMEM((CH, O), jnp.bfloat16) for _ in range(NBUF)],
            [pltpu.SemaphoreType.DMA for _ in range(NBUF)],
            [pltpu.SemaphoreType.DMA for _ in range(NBUF)],
            [pltpu.SemaphoreType.DMA for _ in range(NBUF)],
        ],
    )
    def sc_kernel(y1_hbm, y2_hbm, g1_hbm, g0_hbm, out_hbm,
                  idx1_v, idx0_v, rows1, rows0, acc, sem1, sem0, semo):
        wid = lax.axis_index("s") * NC + lax.axis_index("c")

        pltpu.sync_copy(g1_hbm.at[wid], idx1_v)
        pltpu.sync_copy(g0_hbm.at[wid], idx0_v)

        def start_gathers(ch, b):
            pltpu.async_copy(y1_hbm.at[idx1_v.at[ch]], rows1[b], sem1[b])
            pltpu.async_copy(y2_hbm.at[idx0_v.at[ch]], rows0[b], sem0[b])

        def wait_gathers(b):
            pltpu.make_async_copy(y1_hbm.at[idx1_v.at[0]], rows1[b],
                                  sem1[b]).wait()
            pltpu.make_async_copy(y2_hbm.at[idx0_v.at[0]], rows0[b],
                                  sem0[b]).wait()

        def wait_out(b):
            pltpu.make_async_copy(acc[b], out_hbm.at[pl.ds(0, CH), :],
                                  semo[b]).wait()

        for b in range(NBUF):
            start_gathers(b, b)

        def compute_chunk(b):
            def pair(p, c2):
                row = p * K
                for j in range(NJ):
                    sl = pl.ds(j * LAN, LAN)
                    m = None
                    for k in range(K):
                        v = rows1[b][row + k, sl] + rows0[b][row + k, sl]
                        m = v if m is None else jnp.maximum(m, v)
                    acc[b][p, sl] = jnp.maximum(m, jnp.bfloat16(0))
                return c2

            lax.fori_loop(0, CH, pair, 0, unroll=2)

        def super_step(si, carry):
            for b in range(NBUF):
                ch = si * NBUF + b
                wait_gathers(b)

                @pl.when(ch >= NBUF)
                def _():
                    wait_out(b)

                compute_chunk(b)
                pltpu.async_copy(acc[b], out_hbm.at[pl.ds(wid * PPW + ch * CH, CH), :],
                                 semo[b])

                @pl.when(ch + NBUF < NCH)
                def _():
                    start_gathers(ch + NBUF, b)
            return carry

        lax.fori_loop(0, NCH // NBUF, super_step, 0)
        for b in range(NBUF):
            wait_out(b)

    return sc_kernel(y1f, y2f, g1, g0)


def _transpose_body(y_ref, o_ref):
    o_ref[0] = y_ref[...].astype(jnp.float32).T


def _transpose(y, B, N, O):
    return pl.pallas_call(
        _transpose_body,
        grid=(B,),
        in_specs=[pl.BlockSpec((N, O), lambda i: (i, 0))],
        out_specs=pl.BlockSpec((1, O, N), lambda i: (i, 0, 0)),
        out_shape=jax.ShapeDtypeStruct((B, O, N), jnp.float32),
    )(y)


def kernel(x, edge_index, W, b):
    B, C, N, _ = x.shape
    K = edge_index.shape[-1]
    O = W.shape[0]

    xs = x[:, :, :, 0]
    bias2 = b[None, :]

    idx = edge_index.astype(jnp.int32)
    base = (jnp.arange(B, dtype=jnp.int32) * N)[None, :, None, None]
    gflat = (idx + base).reshape(2, B * N * K)

    y1f, y2f = _make_tables(xs, W, bias2)

    info = plsc.get_sparse_core_info()
    NC, NS = info.num_cores, info.num_subcores
    NW = NC * NS
    PAIRS = B * N
    PPW = PAIRS // NW
    CH = 8
    NCH = PPW // CH
    g = gflat.reshape(2, NW, NCH, CH * K)

    out_f = _edge_max_call(y1f, y2f, g[1], g[0],
                           K=K, O=O, CH=CH, NCH=NCH, PPW=PPW, NC=NC)

    out = _transpose(out_f, B, N, O)
    return out[..., None]


# speedup vs baseline: 30.9446x; 1.0175x over previous
import functools

import jax
import jax.numpy as jnp
from jax import lax
from jax.experimental import pallas as pl
from jax.experimental.pallas import tpu as pltpu
from jax.experimental.pallas import tpu_sc as plsc

_L = 128


def _tables_body(x_ref, w_ref, bias_ref, y1a_ref, y1b_ref, y2a_ref, y2b_ref,
                 *, C, N):
    xb = x_ref[0]
    w1 = w_ref[:, :C]
    w2 = w_ref[:, C:]
    a1 = w1 - w2
    y1 = lax.dot_general(xb, a1, (((0,), (1,)), ((), ())),
                         preferred_element_type=jnp.float32)
    y2 = lax.dot_general(xb, w2, (((0,), (1,)), ((), ())),
                         preferred_element_type=jnp.float32)
    y1 = y1 + bias_ref[0][None, :]
    pad = jnp.zeros((N, 2 * _L - y1.shape[1]), jnp.float32)
    y1a_ref[...] = y1[:, :_L].astype(jnp.bfloat16)
    y1b_ref[...] = jnp.concatenate([y1[:, _L:], pad], 1).astype(jnp.bfloat16)
    y2a_ref[...] = y2[:, :_L].astype(jnp.bfloat16)
    y2b_ref[...] = jnp.concatenate([y2[:, _L:], pad], 1).astype(jnp.bfloat16)


def _make_tables(xs, W, bias2):
    B, C, N = xs.shape
    O = W.shape[0]
    tbl = lambda: jax.ShapeDtypeStruct((B * N, _L), jnp.bfloat16)
    spec = lambda: pl.BlockSpec((N, _L), lambda i: (i, 0))
    return pl.pallas_call(
        functools.partial(_tables_body, C=C, N=N),
        grid=(B,),
        in_specs=[
            pl.BlockSpec((1, C, N), lambda i: (i, 0, 0)),
            pl.BlockSpec((O, 2 * C), lambda i: (0, 0)),
            pl.BlockSpec((1, O), lambda i: (0, 0)),
        ],
        out_specs=[spec(), spec(), spec(), spec()],
        out_shape=[tbl(), tbl(), tbl(), tbl()],
    )(xs, W, bias2)


def _edge_max_call(tables, g1, g0, *, K, O, CH, NCH, PPW, NC):
    R = tables[0].shape[0]
    IDXC = CH * K
    LAN = 32
    NJA = _L // LAN
    NJB = (O - _L) // LAN
    NBUF = 2

    mesh = plsc.VectorSubcoreMesh(core_axis_name="c", subcore_axis_name="s")

    @functools.partial(
        pl.kernel,
        out_type=(jax.ShapeDtypeStruct((R, _L), jnp.bfloat16),
                  jax.ShapeDtypeStruct((R, _L), jnp.bfloat16)),
        mesh=mesh,
        compiler_params=pltpu.CompilerParams(use_tc_tiling_on_sc=False),
        scratch_types=[
            pltpu.VMEM((NCH, IDXC), jnp.int32),
            pltpu.VMEM((NCH, IDXC), jnp.int32),
            [[pltpu.VMEM((IDXC, _L), jnp.bfloat16) for _ in range(4)]
             for _ in range(NBUF)],
            [[pltpu.VMEM((CH, _L), jnp.bfloat16) for _ in range(2)]
             for _ in range(NBUF)],
            [[pltpu.SemaphoreType.DMA for _ in range(4)] for _ in range(NBUF)],
            [[pltpu.SemaphoreType.DMA for _ in range(2)] for _ in range(NBUF)],
        ],
    )
    def sc_kernel(t1a, t1b, t2a, t2b, g1_hbm, g0_hbm, oa_hbm, ob_hbm,
                  idx1_v, idx0_v, rows, acc, semg, semo):
        wid = lax.axis_index("s") * NC + lax.axis_index("c")

        pltpu.sync_copy(g1_hbm.at[pl.ds(wid * NCH, NCH)], idx1_v)
        pltpu.sync_copy(g0_hbm.at[pl.ds(wid * NCH, NCH)], idx0_v)

        def start_gathers(ch, b):
            pltpu.async_copy(t1a.at[idx1_v.at[ch]], rows[b][0], semg[b][0])
            pltpu.async_copy(t1b.at[idx1_v.at[ch]], rows[b][1], semg[b][1])
            pltpu.async_copy(t2a.at[idx0_v.at[ch]], rows[b][2], semg[b][2])
            pltpu.async_copy(t2b.at[idx0_v.at[ch]], rows[b][3], semg[b][3])

        def wait_gathers(b):
            for t in range(4):
                pltpu.make_async_copy(t1a.at[idx1_v.at[0]], rows[b][t],
                                      semg[b][t]).wait()

        def wait_out(b):
            for t in range(2):
                pltpu.make_async_copy(acc[b][t], oa_hbm.at[pl.ds(0, CH), :],
                                      semo[b][t]).wait()

        for b in range(NBUF):
            start_gathers(b, b)

        def compute_chunk(b):
            def pair(p, c2):
                row = p * K
                for half, nj in ((0, NJA), (1, NJB)):
                    r1 = rows[b][half]
                    r0 = rows[b][2 + half]
                    for j in range(nj):
                        sl = pl.ds(j * LAN, LAN)
                        m = None
                        for k in range(K):
                            v = r1[row + k, sl] + r0[row + k, sl]
                            m = v if m is None else jnp.maximum(m, v)
                        acc[b][half][p, sl] = jnp.maximum(m, jnp.bfloat16(0))
                return c2

            lax.fori_loop(0, CH, pair, 0, unroll=2)

        def super_step(si, carry):
            for b in range(NBUF):
                ch = si * NBUF + b
                wait_gathers(b)

                @pl.when(ch >= NBUF)
                def _():
                    wait_out(b)

                compute_chunk(b)
                obase = wid * PPW + ch * CH
                pltpu.async_copy(acc[b][0], oa_hbm.at[pl.ds(obase, CH), :],
                                 semo[b][0])
                pltpu.async_copy(acc[b][1], ob_hbm.at[pl.ds(obase, CH), :],
                                 semo[b][1])

                @pl.when(ch + NBUF < NCH)
                def _():
                    start_gathers(ch + NBUF, b)
            return carry

        lax.fori_loop(0, NCH // NBUF, super_step, 0)
        for b in range(NBUF):
            wait_out(b)

    return sc_kernel(*tables, g1, g0)


def _transpose_body(ya_ref, yb_ref, o_ref, *, O):
    y = jnp.concatenate([ya_ref[...], yb_ref[:, : O - _L]], axis=1)
    o_ref[0] = y.astype(jnp.float32).T


def _transpose(ya, yb, B, N, O):
    return pl.pallas_call(
        functools.partial(_transpose_body, O=O),
        grid=(B,),
        in_specs=[pl.BlockSpec((N, _L), lambda i: (i, 0)),
                  pl.BlockSpec((N, _L), lambda i: (i, 0))],
        out_specs=pl.BlockSpec((1, O, N), lambda i: (i, 0, 0)),
        out_shape=jax.ShapeDtypeStruct((B, O, N), jnp.float32),
    )(ya, yb)


def kernel(x, edge_index, W, b):
    B, C, N, _ = x.shape
    K = edge_index.shape[-1]
    O = W.shape[0]

    xs = x[:, :, :, 0]
    bias2 = b[None, :]

    idx = edge_index.astype(jnp.int32)
    base = (jnp.arange(B, dtype=jnp.int32) * N)[None, :, None, None]
    gflat = (idx + base).reshape(2, B * N * K)

    tables = _make_tables(xs, W, bias2)

    info = plsc.get_sparse_core_info()
    NC, NS = info.num_cores, info.num_subcores
    NW = NC * NS
    PAIRS = B * N
    PPW = PAIRS // NW
    CH = 8
    NCH = PPW // CH
    g = gflat.reshape(2, NW * NCH, CH * K)

    oa, ob = _edge_max_call(tables, g[1], g[0],
                            K=K, O=O, CH=CH, NCH=NCH, PPW=PPW, NC=NC)

    out = _transpose(oa, ob, B, N, O)
    return out[..., None]


# speedup vs baseline: 32.3727x; 1.0462x over previous
import functools

import jax
import jax.numpy as jnp
from jax import lax
from jax.experimental import pallas as pl
from jax.experimental.pallas import tpu as pltpu
from jax.experimental.pallas import tpu_sc as plsc

_L = 128


def _tables_body(x_ref, w_ref, bias_ref, y1a_ref, y1b_ref, y2a_ref, y2b_ref,
                 *, C, N):
    xb = x_ref[0]
    w1 = w_ref[:, :C]
    w2 = w_ref[:, C:]
    a1 = w1 - w2
    y1 = lax.dot_general(xb, a1, (((0,), (1,)), ((), ())),
                         preferred_element_type=jnp.float32)
    y2 = lax.dot_general(xb, w2, (((0,), (1,)), ((), ())),
                         preferred_element_type=jnp.float32)
    y1 = y1 + bias_ref[0][None, :]
    pad = jnp.zeros((N, 2 * _L - y1.shape[1]), jnp.float32)
    y1a_ref[...] = y1[:, :_L].astype(jnp.bfloat16)
    y1b_ref[...] = jnp.concatenate([y1[:, _L:], pad], 1).astype(jnp.bfloat16)
    y2a_ref[...] = y2[:, :_L].astype(jnp.bfloat16)
    y2b_ref[...] = jnp.concatenate([y2[:, _L:], pad], 1).astype(jnp.bfloat16)


def _make_tables(xs, W, bias2, off, Bh):
    _, C, N = xs.shape
    O = W.shape[0]
    tbl = lambda: jax.ShapeDtypeStruct((Bh * N, _L), jnp.bfloat16)
    spec = lambda: pl.BlockSpec((N, _L), lambda i: (i, 0))
    return pl.pallas_call(
        functools.partial(_tables_body, C=C, N=N),
        grid=(Bh,),
        in_specs=[
            pl.BlockSpec((1, C, N), lambda i: (i + off, 0, 0)),
            pl.BlockSpec((O, 2 * C), lambda i: (0, 0)),
            pl.BlockSpec((1, O), lambda i: (0, 0)),
        ],
        out_specs=[spec(), spec(), spec(), spec()],
        out_shape=[tbl(), tbl(), tbl(), tbl()],
    )(xs, W, bias2)


def _edge_max_call(tables, g1, g0, *, K, O, CH, NCH, PPW, NC):
    R = tables[0].shape[0]
    IDXC = CH * K
    LAN = 32
    NJA = _L // LAN
    NJB = (O - _L) // LAN
    NBUF = 2

    mesh = plsc.VectorSubcoreMesh(core_axis_name="c", subcore_axis_name="s")

    @functools.partial(
        pl.kernel,
        out_type=(jax.ShapeDtypeStruct((R, _L), jnp.bfloat16),
                  jax.ShapeDtypeStruct((R, _L), jnp.bfloat16)),
        mesh=mesh,
        compiler_params=pltpu.CompilerParams(use_tc_tiling_on_sc=False),
        scratch_types=[
            pltpu.VMEM((NCH, IDXC), jnp.int32),
            pltpu.VMEM((NCH, IDXC), jnp.int32),
            [[pltpu.VMEM((IDXC, _L), jnp.bfloat16) for _ in range(4)]
             for _ in range(NBUF)],
            [[pltpu.VMEM((CH, _L), jnp.bfloat16) for _ in range(2)]
             for _ in range(NBUF)],
            [[pltpu.SemaphoreType.DMA for _ in range(4)] for _ in range(NBUF)],
            [[pltpu.SemaphoreType.DMA for _ in range(2)] for _ in range(NBUF)],
        ],
    )
    def sc_kernel(t1a, t1b, t2a, t2b, g1_hbm, g0_hbm, oa_hbm, ob_hbm,
                  idx1_v, idx0_v, rows, acc, semg, semo):
        wid = lax.axis_index("s") * NC + lax.axis_index("c")

        pltpu.sync_copy(g1_hbm.at[pl.ds(wid * NCH, NCH)], idx1_v)
        pltpu.sync_copy(g0_hbm.at[pl.ds(wid * NCH, NCH)], idx0_v)

        def start_gathers(ch, b):
            pltpu.async_copy(t1a.at[idx1_v.at[ch]], rows[b][0], semg[b][0])
            pltpu.async_copy(t1b.at[idx1_v.at[ch]], rows[b][1], semg[b][1])
            pltpu.async_copy(t2a.at[idx0_v.at[ch]], rows[b][2], semg[b][2])
            pltpu.async_copy(t2b.at[idx0_v.at[ch]], rows[b][3], semg[b][3])

        def wait_gathers(b):
            for t in range(4):
                pltpu.make_async_copy(t1a.at[idx1_v.at[0]], rows[b][t],
                                      semg[b][t]).wait()

        def wait_out(b):
            for t in range(2):
                pltpu.make_async_copy(acc[b][t], oa_hbm.at[pl.ds(0, CH), :],
                                      semo[b][t]).wait()

        for b in range(NBUF):
            start_gathers(b, b)

        def compute_chunk(b):
            def pair(p, c2):
                row = p * K
                for half, nj in ((0, NJA), (1, NJB)):
                    r1 = rows[b][half]
                    r0 = rows[b][2 + half]
                    for j in range(nj):
                        sl = pl.ds(j * LAN, LAN)
                        m = None
                        for k in range(K):
                            v = r1[row + k, sl] + r0[row + k, sl]
                            m = v if m is None else jnp.maximum(m, v)
                        acc[b][half][p, sl] = jnp.maximum(m, jnp.bfloat16(0))
                return c2

            lax.fori_loop(0, CH, pair, 0, unroll=2)

        def super_step(si, carry):
            for b in range(NBUF):
                ch = si * NBUF + b
                wait_gathers(b)

                @pl.when(ch >= NBUF)
                def _():
                    wait_out(b)

                compute_chunk(b)
                obase = wid * PPW + ch * CH
                pltpu.async_copy(acc[b][0], oa_hbm.at[pl.ds(obase, CH), :],
                                 semo[b][0])
                pltpu.async_copy(acc[b][1], ob_hbm.at[pl.ds(obase, CH), :],
                                 semo[b][1])

                @pl.when(ch + NBUF < NCH)
                def _():
                    start_gathers(ch + NBUF, b)
            return carry

        lax.fori_loop(0, NCH // NBUF, super_step, 0)
        for b in range(NBUF):
            wait_out(b)

    return sc_kernel(*tables, g1, g0)


def _transpose_body(ya1_ref, yb1_ref, ya2_ref, yb2_ref, o_ref, *, O):
    h = pl.program_id(0)

    @pl.when(h == 0)
    def _():
        y = jnp.concatenate([ya1_ref[...], yb1_ref[:, : O - _L]], axis=1)
        o_ref[0] = y.astype(jnp.float32).T

    @pl.when(h == 1)
    def _():
        y = jnp.concatenate([ya2_ref[...], yb2_ref[:, : O - _L]], axis=1)
        o_ref[0] = y.astype(jnp.float32).T


def _transpose(halves, B, N, O):
    Bh = B // 2
    s1 = lambda: pl.BlockSpec((N, _L), lambda h, i: ((1 - h) * i, 0))
    s2 = lambda: pl.BlockSpec((N, _L), lambda h, i: (h * i, 0))
    return pl.pallas_call(
        functools.partial(_transpose_body, O=O),
        grid=(2, Bh),
        in_specs=[s1(), s1(), s2(), s2()],
        out_specs=pl.BlockSpec((1, O, N), lambda h, i: (h * Bh + i, 0, 0)),
        out_shape=jax.ShapeDtypeStruct((B, O, N), jnp.float32),
    )(*halves)


def kernel(x, edge_index, W, b):
    B, C, N, _ = x.shape
    K = edge_index.shape[-1]
    O = W.shape[0]

    xs = x[:, :, :, 0]
    bias2 = b[None, :]
    idx = edge_index.astype(jnp.int32)

    info = plsc.get_sparse_core_info()
    NC, NS = info.num_cores, info.num_subcores
    NW = NC * NS
    Bh = B // 2
    PAIRS = Bh * N
    PPW = PAIRS // NW
    CH = 8
    NCH = PPW // CH

    outs = []
    for h in range(2):
        base = (jnp.arange(Bh, dtype=jnp.int32) * N)[None, :, None, None]
        g = (idx[:, h * Bh:(h + 1) * Bh] + base).reshape(2, NW * NCH, CH * K)
        tables = _make_tables(xs, W, bias2, h * Bh, Bh)
        outs.append(_edge_max_call(tables, g[1], g[0], K=K, O=O, CH=CH,
                                   NCH=NCH, PPW=PPW, NC=NC))

    out = _transpose([*outs[0], *outs[1]], B, N, O)
    return out[..., None]


# speedup vs baseline: 38.0471x; 1.1753x over previous
import functools

import jax
import jax.numpy as jnp
from jax import lax
from jax.experimental import pallas as pl
from jax.experimental.pallas import tpu as pltpu
from jax.experimental.pallas import tpu_sc as plsc

_L = 128


def _enc(y):
    n, o = y.shape
    h = o // 2
    yb = y.astype(jnp.bfloat16)
    lo = lax.bitcast_convert_type(yb[:, :h], jnp.uint16).astype(jnp.uint32)
    hi = lax.bitcast_convert_type(yb[:, h:], jnp.uint16).astype(jnp.uint32)
    w = lo | (hi << 16)
    pad = jnp.zeros((n, _L - h), jnp.uint32)
    return jnp.concatenate([w, pad], axis=1)


def _dec(w, O):
    h = O // 2
    w = w[:, :h]
    lo = lax.bitcast_convert_type((w & 0xFFFF).astype(jnp.uint16),
                                  jnp.bfloat16)
    hi = lax.bitcast_convert_type((w >> 16).astype(jnp.uint16),
                                  jnp.bfloat16)
    return jnp.concatenate([lo, hi], axis=1).astype(jnp.float32)


def _tables_body(x_ref, w_ref, bias_ref, t1_ref, t2_ref, *, C):
    xb = x_ref[0]
    w1 = w_ref[:, :C]
    w2 = w_ref[:, C:]
    a1 = w1 - w2
    y1 = lax.dot_general(xb, a1, (((0,), (1,)), ((), ())),
                         preferred_element_type=jnp.float32)
    y2 = lax.dot_general(xb, w2, (((0,), (1,)), ((), ())),
                         preferred_element_type=jnp.float32)
    t1_ref[...] = _enc(y1 + bias_ref[0][None, :])
    t2_ref[...] = _enc(y2)


def _make_tables(xs, W, bias2, off, Bh):
    _, C, N = xs.shape
    O = W.shape[0]
    tbl = lambda: jax.ShapeDtypeStruct((Bh * N, _L), jnp.uint32)
    spec = lambda: pl.BlockSpec((N, _L), lambda i: (i, 0))
    return pl.pallas_call(
        functools.partial(_tables_body, C=C),
        grid=(Bh,),
        in_specs=[
            pl.BlockSpec((1, C, N), lambda i: (i + off, 0, 0)),
            pl.BlockSpec((O, 2 * C), lambda i: (0, 0)),
            pl.BlockSpec((1, O), lambda i: (0, 0)),
        ],
        out_specs=[spec(), spec()],
        out_shape=[tbl(), tbl()],
    )(xs, W, bias2)


def _edge_max_call(t1f, t2f, g1, g0, *, K, CH, NCH, PPW, NC):
    R = t1f.shape[0]
    IDXC = CH * K
    NIW = PPW * K
    NJ = _L // 16
    NJU = (_L - 32) // 16
    NBUF = 2

    mesh = plsc.VectorSubcoreMesh(core_axis_name="c", subcore_axis_name="s")

    @functools.partial(
        pl.kernel,
        out_type=jax.ShapeDtypeStruct((R, _L), jnp.uint32),
        mesh=mesh,
        compiler_params=pltpu.CompilerParams(needs_layout_passes=False),
        scratch_types=[
            pltpu.VMEM((NIW,), jnp.int32),
            pltpu.VMEM((NIW,), jnp.int32),
            [[pltpu.VMEM((IDXC, _L), jnp.uint32) for _ in range(2)]
             for _ in range(NBUF)],
            [pltpu.VMEM((CH, _L), jnp.uint32) for _ in range(NBUF)],
            [[pltpu.SemaphoreType.DMA for _ in range(2)] for _ in range(NBUF)],
            [pltpu.SemaphoreType.DMA for _ in range(NBUF)],
        ],
    )
    def sc_kernel(t1, t2, g1_hbm, g0_hbm, out_hbm,
                  idx1_v, idx0_v, rows, acc, semg, semo):
        wid = lax.axis_index("s") * NC + lax.axis_index("c")

        pltpu.sync_copy(g1_hbm.at[pl.ds(wid * NIW, NIW)], idx1_v)
        pltpu.sync_copy(g0_hbm.at[pl.ds(wid * NIW, NIW)], idx0_v)

        def start_gathers(ch, b):
            i1 = idx1_v.at[pl.ds(ch * IDXC, IDXC)]
            i0 = idx0_v.at[pl.ds(ch * IDXC, IDXC)]
            pltpu.async_copy(t1.at[i1], rows[b][0], semg[b][0])
            pltpu.async_copy(t2.at[i0], rows[b][1], semg[b][1])

        def wait_gathers(b):
            for t in range(2):
                pltpu.make_async_copy(t1.at[idx1_v.at[pl.ds(0, IDXC)]],
                                      rows[b][t], semg[b][t]).wait()

        def wait_out(b):
            pltpu.make_async_copy(acc[b], out_hbm.at[pl.ds(0, CH), :],
                                  semo[b]).wait()

        for b in range(NBUF):
            start_gathers(b, b)

        def compute_chunk(b):
            r1 = rows[b][0]
            r0 = rows[b][1]

            def pair(p, c2):
                row = p * K
                for j in range(NJU):
                    sl = pl.ds(j * 16, 16)
                    m = None
                    for k in range(K):
                        v1 = plsc.bitcast(r1[row + k, sl], jnp.bfloat16)
                        v0 = plsc.bitcast(r0[row + k, sl], jnp.bfloat16)
                        v = v1 + v0
                        m = v if m is None else jnp.maximum(m, v)
                    m = jnp.maximum(m, jnp.bfloat16(0))
                    acc[b][p, sl] = plsc.bitcast(m, jnp.uint32)
                return c2

            lax.fori_loop(0, CH, pair, 0, unroll=2)

        def super_step(si, carry):
            for b in range(NBUF):
                ch = si * NBUF + b
                wait_gathers(b)

                @pl.when(ch >= NBUF)
                def _():
                    wait_out(b)

                compute_chunk(b)
                pltpu.async_copy(acc[b],
                                 out_hbm.at[pl.ds(wid * PPW + ch * CH, CH), :],
                                 semo[b])

                @pl.when(ch + NBUF < NCH)
                def _():
                    start_gathers(ch + NBUF, b)
            return carry

        lax.fori_loop(0, NCH // NBUF, super_step, 0)
        for b in range(NBUF):
            wait_out(b)

    return sc_kernel(t1f, t2f, g1, g0)


def _transpose_body(o1_ref, o2_ref, o_ref, *, O):
    h = pl.program_id(0)

    @pl.when(h == 0)
    def _():
        o_ref[0] = _dec(o1_ref[...], O).T

    @pl.when(h == 1)
    def _():
        o_ref[0] = _dec(o2_ref[...], O).T


def _transpose(halves, B, N, O):
    Bh = B // 2
    s1 = pl.BlockSpec((N, _L), lambda h, i: ((1 - h) * i, 0))
    s2 = pl.BlockSpec((N, _L), lambda h, i: (h * i, 0))
    return pl.pallas_call(
        functools.partial(_transpose_body, O=O),
        grid=(2, Bh),
        in_specs=[s1, s2],
        out_specs=pl.BlockSpec((1, O, N), lambda h, i: (h * Bh + i, 0, 0)),
        out_shape=jax.ShapeDtypeStruct((B, O, N), jnp.float32),
    )(*halves)


def kernel(x, edge_index, W, b):
    B, C, N, _ = x.shape
    K = edge_index.shape[-1]
    O = W.shape[0]

    xs = x[:, :, :, 0]
    bias2 = b[None, :]
    idx = edge_index.astype(jnp.int32)

    info = plsc.get_sparse_core_info()
    NC, NS = info.num_cores, info.num_subcores
    NW = NC * NS
    Bh = B // 2
    PAIRS = Bh * N
    PPW = PAIRS // NW
    CH = 8
    NCH = PPW // CH

    outs = []
    for h in range(2):
        base = (jnp.arange(Bh, dtype=jnp.int32) * N)[None, :, None, None]
        g = (idx[:, h * Bh:(h + 1) * Bh] + base).reshape(2, Bh * N * K)
        t1, t2 = _make_tables(xs, W, bias2, h * Bh, Bh)
        outs.append(_edge_max_call(t1, t2, g[1], g[0], K=K, CH=CH,
                                   NCH=NCH, PPW=PPW, NC=NC))

    out = _transpose(outs, B, N, O)
    return out[..., None]


# speedup vs baseline: 38.7487x; 1.0184x over previous
import functools

import jax
import jax.numpy as jnp
from jax import lax
from jax.experimental import pallas as pl
from jax.experimental.pallas import tpu as pltpu
from jax.experimental.pallas import tpu_sc as plsc

_L = 128


def _enc(y):
    n, o = y.shape
    h = o // 2
    yb = y.astype(jnp.bfloat16)
    lo = lax.bitcast_convert_type(yb[:, :h], jnp.uint16).astype(jnp.uint32)
    hi = lax.bitcast_convert_type(yb[:, h:], jnp.uint16).astype(jnp.uint32)
    w = lo | (hi << 16)
    pad = jnp.zeros((n, _L - h), jnp.uint32)
    return jnp.concatenate([w, pad], axis=1)


def _dec(w, O):
    h = O // 2
    w = w[:, :h]
    lo = lax.bitcast_convert_type((w & 0xFFFF).astype(jnp.uint16),
                                  jnp.bfloat16)
    hi = lax.bitcast_convert_type((w >> 16).astype(jnp.uint16),
                                  jnp.bfloat16)
    return jnp.concatenate([lo, hi], axis=1).astype(jnp.float32)


def _tables_body(x_ref, w_ref, bias_ref, t1_ref, t2_ref, *, C):
    xb = x_ref[0]
    w1 = w_ref[:, :C]
    w2 = w_ref[:, C:]
    a1 = w1 - w2
    y1 = lax.dot_general(xb, a1, (((0,), (1,)), ((), ())),
                         preferred_element_type=jnp.float32)
    y2 = lax.dot_general(xb, w2, (((0,), (1,)), ((), ())),
                         preferred_element_type=jnp.float32)
    t1_ref[...] = _enc(y1 + bias_ref[0][None, :])
    t2_ref[...] = _enc(y2)


def _make_tables(xs, W, bias2, off, Bh):
    _, C, N = xs.shape
    O = W.shape[0]
    tbl = lambda: jax.ShapeDtypeStruct((Bh * N, _L), jnp.uint32)
    spec = lambda: pl.BlockSpec((N, _L), lambda i: (i, 0))
    return pl.pallas_call(
        functools.partial(_tables_body, C=C),
        grid=(Bh,),
        in_specs=[
            pl.BlockSpec((1, C, N), lambda i: (i + off, 0, 0)),
            pl.BlockSpec((O, 2 * C), lambda i: (0, 0)),
            pl.BlockSpec((1, O), lambda i: (0, 0)),
        ],
        out_specs=[spec(), spec()],
        out_shape=[tbl(), tbl()],
    )(xs, W, bias2)


def _edge_max_call(t1f, t2f, g1, g0, *, K, CH, NCH, PPW, NC):
    R = t1f.shape[0]
    IDXC = CH * K
    NIW = PPW * K
    NJ = _L // 16
    NJU = (_L - 32) // 16
    NBUF = 3

    mesh = plsc.VectorSubcoreMesh(core_axis_name="c", subcore_axis_name="s")

    @functools.partial(
        pl.kernel,
        out_type=jax.ShapeDtypeStruct((R, _L), jnp.uint32),
        mesh=mesh,
        compiler_params=pltpu.CompilerParams(needs_layout_passes=False),
        scratch_types=[
            pltpu.VMEM((NIW,), jnp.int32),
            pltpu.VMEM((NIW,), jnp.int32),
            [[pltpu.VMEM((IDXC, _L), jnp.uint32) for _ in range(2)]
             for _ in range(NBUF)],
            [pltpu.VMEM((CH, _L), jnp.uint32) for _ in range(NBUF)],
            [[pltpu.SemaphoreType.DMA for _ in range(2)] for _ in range(NBUF)],
            [pltpu.SemaphoreType.DMA for _ in range(NBUF)],
        ],
    )
    def sc_kernel(t1, t2, g1_hbm, g0_hbm, out_hbm,
                  idx1_v, idx0_v, rows, acc, semg, semo):
        wid = lax.axis_index("s") * NC + lax.axis_index("c")

        pltpu.sync_copy(g1_hbm.at[pl.ds(wid * NIW, NIW)], idx1_v)
        pltpu.sync_copy(g0_hbm.at[pl.ds(wid * NIW, NIW)], idx0_v)

        def start_gathers(ch, b):
            i1 = idx1_v.at[pl.ds(ch * IDXC, IDXC)]
            i0 = idx0_v.at[pl.ds(ch * IDXC, IDXC)]
            pltpu.async_copy(t1.at[i1], rows[b][0], semg[b][0])
            pltpu.async_copy(t2.at[i0], rows[b][1], semg[b][1])

        def wait_gathers(b):
            for t in range(2):
                pltpu.make_async_copy(t1.at[idx1_v.at[pl.ds(0, IDXC)]],
                                      rows[b][t], semg[b][t]).wait()

        def wait_out(b):
            pltpu.make_async_copy(acc[b], out_hbm.at[pl.ds(0, CH), :],
                                  semo[b]).wait()

        for b in range(NBUF):
            start_gathers(b, b)

        def compute_chunk(b):
            r1 = rows[b][0]
            r0 = rows[b][1]

            def pair(p, c2):
                row = p * K
                for j in range(NJU):
                    sl = pl.ds(j * 16, 16)
                    m = None
                    for k in range(K):
                        v1 = plsc.bitcast(r1[row + k, sl], jnp.bfloat16)
                        v0 = plsc.bitcast(r0[row + k, sl], jnp.bfloat16)
                        v = v1 + v0
                        m = v if m is None else jnp.maximum(m, v)
                    m = jnp.maximum(m, jnp.bfloat16(0))
                    acc[b][p, sl] = plsc.bitcast(m, jnp.uint32)
                return c2

            lax.fori_loop(0, CH, pair, 0, unroll=2)

        def super_step(si, carry):
            for b in range(NBUF):
                ch = si * NBUF + b
                wait_gathers(b)

                @pl.when(ch >= NBUF)
                def _():
                    wait_out(b)

                compute_chunk(b)
                pltpu.async_copy(acc[b],
                                 out_hbm.at[pl.ds(wid * PPW + ch * CH, CH), :],
                                 semo[b])

                @pl.when(ch + NBUF < NCH)
                def _():
                    start_gathers(ch + NBUF, b)
            return carry

        lax.fori_loop(0, NCH // NBUF, super_step, 0)
        for b in range(NBUF):
            wait_out(b)

    return sc_kernel(t1f, t2f, g1, g0)


def _transpose_body(o1_ref, o2_ref, o_ref, *, O):
    h = pl.program_id(0)

    @pl.when(h == 0)
    def _():
        o_ref[0] = _dec(o1_ref[...], O).T

    @pl.when(h == 1)
    def _():
        o_ref[0] = _dec(o2_ref[...], O).T


def _transpose(halves, B, N, O):
    Bh = B // 2
    s1 = pl.BlockSpec((N, _L), lambda h, i: ((1 - h) * i, 0))
    s2 = pl.BlockSpec((N, _L), lambda h, i: (h * i, 0))
    return pl.pallas_call(
        functools.partial(_transpose_body, O=O),
        grid=(2, Bh),
        in_specs=[s1, s2],
        out_specs=pl.BlockSpec((1, O, N), lambda h, i: (h * Bh + i, 0, 0)),
        out_shape=jax.ShapeDtypeStruct((B, O, N), jnp.float32),
    )(*halves)


def kernel(x, edge_index, W, b):
    B, C, N, _ = x.shape
    K = edge_index.shape[-1]
    O = W.shape[0]

    xs = x[:, :, :, 0]
    bias2 = b[None, :]
    idx = edge_index.astype(jnp.int32)

    info = plsc.get_sparse_core_info()
    NC, NS = info.num_cores, info.num_subcores
    NW = NC * NS
    Bh = B // 2
    PAIRS = Bh * N
    PPW = PAIRS // NW
    CH = 8
    NCH = PPW // CH

    outs = []
    for h in range(2):
        base = (jnp.arange(Bh, dtype=jnp.int32) * N)[None, :, None, None]
        g = (idx[:, h * Bh:(h + 1) * Bh] + base).reshape(2, Bh * N * K)
        t1, t2 = _make_tables(xs, W, bias2, h * Bh, Bh)
        outs.append(_edge_max_call(t1, t2, g[1], g[0], K=K, CH=CH,
                                   NCH=NCH, PPW=PPW, NC=NC))

    out = _transpose(outs, B, N, O)
    return out[..., None]
